# Initial kernel scaffold; baseline (speedup 1.0000x reference)
import jax
import jax.numpy as jnp
from jax.experimental import pallas as pl


def kernel(sentence1, sentence2, label, embed_table, bn1_gamma, bn1_beta, fc1_w, fc1_b, bn2_gamma, bn2_beta, fc2_w, fc2_b):
    raise NotImplementedError("write your pallas kernel here")


# speedup vs baseline: 1.0634x; 1.0634x over previous
import jax
import jax.numpy as jnp
from jax import lax
from jax.experimental import pallas as pl
from jax.experimental.pallas import tpu as pltpu
from jax.experimental.pallas import tpu_sc as plsc

D_EMBED = 64
D_PAD = 128
SEQ = 200
BATCH = 4096

_NC = 2
_NS = 16
_NW = _NC * _NS
_COLS = 2 * BATCH
_BPW = _COLS // _NW
_CH = 128
_NCHUNK = _BPW // _CH
_NBANK = 4


def _pool_body(s_hbm, table_hbm, out_hbm, idx_v, acc_v, sems):
    wid = lax.axis_index("s") * _NC + lax.axis_index("c")

    pltpu.sync_copy(s_hbm.at[pl.ds(_NCHUNK * wid, _NCHUNK)], idx_v)

    zeros = jnp.zeros((16,), jnp.float32)

    def zrow(r, c):
        for b in range(_NBANK):
            for k in range(D_PAD // 16):
                acc_v[b, r, pl.ds(k * 16, 16)] = zeros
        return c

    lax.fori_loop(0, _CH, zrow, 0)

    def fire(s, b):
        return pltpu.async_copy(
            table_hbm.at[idx_v.at[b % 2, s]], acc_v.at[b], sems.at[b],
            add=True)

    def wait(s, b):
        pltpu.make_async_copy(
            table_hbm.at[idx_v.at[b % 2, s]], acc_v.at[b], sems.at[b]).wait()

    for b in range(_NBANK):
        fire(b // 2, b)

    def step(u, c):
        s = 2 * u
        for b in range(_NBANK):
            wait(s - 2 + b // 2, b)
            fire(s + b // 2, b)
        return c

    lax.fori_loop(1, SEQ // 2, step, 0)

    for b in range(_NBANK):
        wait(SEQ - 2 + b // 2, b)

    def mrow(r, c):
        for b in range(2):
            for k in range(D_PAD // 16):
                v = acc_v[b + 2, r, pl.ds(k * 16, 16)]
                plsc.addupdate(acc_v.at[b, r, pl.ds(k * 16, 16)], v)
        return c

    lax.fori_loop(0, _CH, mrow, 0)

    pltpu.sync_copy(acc_v.at[pl.ds(0, 2)],
                    out_hbm.at[pl.ds(_NCHUNK * wid, _NCHUNK)])


_PREP_ROWS = 8000


def _prep_body(t_ref, out_ref):
    x = t_ref[...]
    out_ref[...] = jnp.concatenate([x, x], axis=1)


def _mlp_body(p_ref, bn1g, bn1b, f1w, f1b, bn2g, bn2b, f2w, f2b, out_ref):
    inv = jnp.float32(1.0 / SEQ)
    x = jnp.concatenate([p_ref[0] * inv, p_ref[1] * inv], axis=1)

    def bn(t, g, b):
        m = jnp.mean(t, axis=0, keepdims=True)
        tc = t - m
        v = jnp.mean(tc * tc, axis=0, keepdims=True)
        return tc / jnp.sqrt(v + 1e-5) * g + b

    x = bn(x, bn1g[...], bn1b[...])
    h = lax.dot_general(x, f1w[...], (((1,), (1,)), ((), ())),
                        preferred_element_type=jnp.float32) + f1b[...]
    h = bn(h, bn2g[...], bn2b[...])
    o = lax.dot_general(h, f2w[...], (((1,), (1,)), ((), ())),
                        preferred_element_type=jnp.float32) + f2b[...]
    out_ref[...] = o


def kernel(sentence1, sentence2, label, embed_table, bn1_gamma, bn1_beta,
           fc1_w, fc1_b, bn2_gamma, bn2_beta, fc2_w, fc2_b):
    del label
    s_all = jnp.concatenate([sentence1, sentence2], axis=1)
    s_all = s_all.T.reshape(_COLS // _CH, _CH, SEQ).transpose(0, 2, 1)

    n_embed = embed_table.shape[0]
    table128 = pl.pallas_call(
        _prep_body,
        grid=(n_embed // _PREP_ROWS,),
        in_specs=[pl.BlockSpec((_PREP_ROWS, D_EMBED), lambda i: (i, 0))],
        out_specs=pl.BlockSpec((_PREP_ROWS, D_PAD), lambda i: (i, 0)),
        out_shape=jax.ShapeDtypeStruct((n_embed, D_PAD), jnp.float32),
    )(embed_table)

    mesh = plsc.VectorSubcoreMesh(core_axis_name="c", subcore_axis_name="s")
    pool = pl.kernel(
        _pool_body,
        mesh=mesh,
        out_type=jax.ShapeDtypeStruct((_COLS // _CH, _CH, D_PAD),
                                      jnp.float32),
        scratch_types=[
            pltpu.VMEM((_NCHUNK, SEQ, _CH), jnp.int32),
            pltpu.VMEM((_NBANK, _CH, D_PAD), jnp.float32),
            pltpu.SemaphoreType.DMA((_NBANK,)),
        ],
    )
    pooled = pool(s_all, table128)

    p3 = pooled.reshape(2, BATCH, D_PAD)[:, :, :D_EMBED]
    out = pl.pallas_call(
        _mlp_body,
        out_shape=jax.ShapeDtypeStruct((BATCH, 2), jnp.float32),
    )(p3, bn1_gamma, bn1_beta, fc1_w, fc1_b, bn2_gamma, bn2_beta, fc2_w,
      fc2_b)
    return out
